# baseline (speedup 1.0000x reference)
import functools

import jax
import jax.numpy as jnp
from jax import lax
from jax.experimental import pallas as pl
from jax.experimental.pallas import tpu as pltpu
from jax.experimental.pallas import tpu_sc as plsc

_CHUNK = 128


def _sc_geometry():
    try:
        info = plsc.get_sparse_core_info()
        return info.num_cores, info.num_subcores
    except Exception:
        return 2, 16


@functools.lru_cache(maxsize=None)
def _build_gather(V, D, B, NC, NS):
    NW = NC * NS
    per_w = B // NW
    n_chunks = per_w // _CHUNK
    mesh = plsc.VectorSubcoreMesh(core_axis_name="c", subcore_axis_name="s")

    @functools.partial(
        pl.kernel,
        mesh=mesh,
        out_type=jax.ShapeDtypeStruct((B, D), jnp.float32),
        scratch_types=[
            pltpu.VMEM((n_chunks, _CHUNK), jnp.int32),
            pltpu.VMEM((_CHUNK, D), jnp.float32),
            pltpu.VMEM((_CHUNK, D), jnp.float32),
            pltpu.SemaphoreType.DMA,
            pltpu.SemaphoreType.DMA,
            pltpu.SemaphoreType.DMA,
        ],
    )
    def gather_kernel(table_hbm, idx_hbm, out_hbm, idx_v, buf0, buf1,
                      gsem, ssem0, ssem1):
        wid = lax.axis_index("s") * NC + lax.axis_index("c")
        base = wid * per_w
        pltpu.sync_copy(idx_hbm.at[wid], idx_v)
        bufs = (buf0, buf1)
        ssems = (ssem0, ssem1)
        pending = [None, None]
        for j in range(n_chunks):
            b = j % 2
            if pending[b] is not None:
                pending[b].wait()
            pltpu.async_copy(table_hbm.at[idx_v.at[j]], bufs[b], gsem).wait()
            pending[b] = pltpu.async_copy(
                bufs[b], out_hbm.at[pl.ds(base + j * _CHUNK, _CHUNK)], ssems[b])
        for p in pending:
            if p is not None:
                p.wait()

    return gather_kernel


def kernel(x, cluster, depth):
    B0 = cluster.shape[0]
    V, D = x.shape
    NC, NS = _sc_geometry()
    NW = NC * NS
    tile = NW * _CHUNK
    B = ((B0 + tile - 1) // tile) * tile
    idx = cluster.astype(jnp.int32)
    if B != B0:
        idx = jnp.concatenate([idx, jnp.zeros((B - B0,), jnp.int32)])
    idx = idx.reshape(NW, B // tile, _CHUNK)
    fn = _build_gather(V, D, B, NC, NS)
    out = lax.cond(
        depth != 0,
        lambda: fn(x, idx),
        lambda: jnp.zeros((B, D), jnp.float32),
    )
    return out[:B0]


# speedup vs baseline: 3.4532x; 3.4532x over previous
import functools

import jax
import jax.numpy as jnp
from jax import lax
from jax.experimental import pallas as pl
from jax.experimental.pallas import tpu as pltpu
from jax.experimental.pallas import tpu_sc as plsc

_CHUNK = 128


def _sc_geometry():
    try:
        info = plsc.get_sparse_core_info()
        return info.num_cores, info.num_subcores
    except Exception:
        return 2, 16


@functools.lru_cache(maxsize=None)
def _build_gather(V, D, B0, n_chunks, NC, NS):
    NW = NC * NS
    per_w = n_chunks // NW
    last_base = B0 - _CHUNK
    mesh = plsc.VectorSubcoreMesh(core_axis_name="c", subcore_axis_name="s")

    @functools.partial(
        pl.kernel,
        mesh=mesh,
        out_type=jax.ShapeDtypeStruct((B0, D), jnp.float32),
        scratch_types=[
            pltpu.VMEM((per_w, _CHUNK), jnp.int32),
            pltpu.VMEM((_CHUNK, D), jnp.float32),
            pltpu.VMEM((_CHUNK, D), jnp.float32),
            pltpu.SemaphoreType.DMA,
            pltpu.SemaphoreType.DMA,
            pltpu.SemaphoreType.DMA,
        ],
    )
    def gather_kernel(table_hbm, idx_hbm, out_hbm, idx_v, buf0, buf1,
                      gsem, ssem0, ssem1):
        wid = lax.axis_index("s") * NC + lax.axis_index("c")
        pltpu.sync_copy(idx_hbm.at[wid], idx_v)
        bufs = (buf0, buf1)
        ssems = (ssem0, ssem1)
        pending = [None, None]
        for j in range(per_w):
            b = j % 2
            if pending[b] is not None:
                pending[b].wait()
            pltpu.async_copy(table_hbm.at[idx_v.at[j]], bufs[b], gsem).wait()
            dst = jnp.minimum((wid * per_w + j) * _CHUNK, last_base)
            pending[b] = pltpu.async_copy(
                bufs[b], out_hbm.at[pl.ds(dst, _CHUNK)], ssems[b])
        for p in pending:
            if p is not None:
                p.wait()

    return gather_kernel


def kernel(x, cluster, depth):
    B0 = cluster.shape[0]
    V, D = x.shape
    NC, NS = _sc_geometry()
    NW = NC * NS
    n_full = B0 // _CHUNK
    n_real = -(-B0 // _CHUNK)
    n_chunks = -(-n_real // NW) * NW
    idx = cluster.astype(jnp.int32)
    parts = [idx[: n_full * _CHUNK].reshape(n_full, _CHUNK)]
    n_tail = n_chunks - n_full
    if n_tail:
        parts.append(jnp.tile(idx[B0 - _CHUNK:][None, :], (n_tail, 1)))
    idx2 = jnp.concatenate(parts).reshape(NW, n_chunks // NW, _CHUNK)
    fn = _build_gather(V, D, B0, n_chunks, NC, NS)
    out = lax.cond(
        depth != 0,
        lambda: fn(x, idx2),
        lambda: jnp.zeros((B0, D), jnp.float32),
    )
    return out


# speedup vs baseline: 3.5476x; 1.0274x over previous
import functools

import jax
import jax.numpy as jnp
from jax import lax
from jax.experimental import pallas as pl
from jax.experimental.pallas import tpu as pltpu
from jax.experimental.pallas import tpu_sc as plsc

_CHUNK = 128


def _sc_geometry():
    try:
        info = plsc.get_sparse_core_info()
        return info.num_cores, info.num_subcores
    except Exception:
        return 2, 16


@functools.lru_cache(maxsize=None)
def _build_gather(V, D, B0, n_chunks, NC, NS):
    NW = NC * NS
    per_w = n_chunks // NW
    span = per_w * _CHUNK
    last_base = B0 - _CHUNK
    last_span = B0 - span
    mesh = plsc.VectorSubcoreMesh(core_axis_name="c", subcore_axis_name="s")

    @functools.partial(
        pl.kernel,
        mesh=mesh,
        out_type=jax.ShapeDtypeStruct((B0, D), jnp.float32),
        scratch_types=[
            pltpu.VMEM((span,), jnp.int32),
            pltpu.VMEM((_CHUNK, D), jnp.float32),
            pltpu.VMEM((_CHUNK, D), jnp.float32),
            pltpu.SemaphoreType.DMA,
            pltpu.SemaphoreType.DMA,
            pltpu.SemaphoreType.DMA,
        ],
    )
    def gather_kernel(table_hbm, idx_hbm, out_hbm, idx_v, buf0, buf1,
                      gsem, ssem0, ssem1):
        wid = lax.axis_index("s") * NC + lax.axis_index("c")
        src0 = jnp.minimum(wid * span, last_span)
        pltpu.sync_copy(idx_hbm.at[pl.ds(src0, span)], idx_v)
        bufs = (buf0, buf1)
        ssems = (ssem0, ssem1)
        pending = [None, None]
        for j in range(per_w):
            b = j % 2
            if pending[b] is not None:
                pending[b].wait()
            dst = jnp.minimum(wid * span + j * _CHUNK, last_base)
            idx_chunk = idx_v.at[pl.ds(dst - src0, _CHUNK)]
            pltpu.async_copy(table_hbm.at[idx_chunk], bufs[b], gsem).wait()
            pending[b] = pltpu.async_copy(
                bufs[b], out_hbm.at[pl.ds(dst, _CHUNK)], ssems[b])
        for p in pending:
            if p is not None:
                p.wait()

    return gather_kernel


def kernel(x, cluster, depth):
    B0 = cluster.shape[0]
    V, D = x.shape
    NC, NS = _sc_geometry()
    NW = NC * NS
    n_real = -(-B0 // _CHUNK)
    n_chunks = -(-n_real // NW) * NW
    idx = cluster.astype(jnp.int32)
    fn = _build_gather(V, D, B0, n_chunks, NC, NS)
    out = lax.cond(
        depth != 0,
        lambda: fn(x, idx),
        lambda: jnp.zeros((B0, D), jnp.float32),
    )
    return out


# speedup vs baseline: 4.0421x; 1.1394x over previous
import functools

import jax
import jax.numpy as jnp
from jax import lax
from jax.experimental import pallas as pl
from jax.experimental.pallas import tpu as pltpu
from jax.experimental.pallas import tpu_sc as plsc

_CHUNK = 256
_NBUF = 3


def _sc_geometry():
    try:
        info = plsc.get_sparse_core_info()
        return info.num_cores, info.num_subcores
    except Exception:
        return 2, 16


@functools.lru_cache(maxsize=None)
def _build_gather(V, D, B0, n_chunks, NC, NS):
    NW = NC * NS
    per_w = n_chunks // NW
    span = per_w * _CHUNK
    last_base = B0 - _CHUNK
    last_span = B0 - span
    mesh = plsc.VectorSubcoreMesh(core_axis_name="c", subcore_axis_name="s")

    @functools.partial(
        pl.kernel,
        mesh=mesh,
        out_type=jax.ShapeDtypeStruct((B0, D), jnp.float32),
        scratch_types=(
            [pltpu.VMEM((span,), jnp.int32)]
            + [pltpu.VMEM((_CHUNK, D), jnp.float32) for _ in range(_NBUF)]
            + [pltpu.SemaphoreType.DMA for _ in range(2 * _NBUF)]
        ),
    )
    def gather_kernel(table_hbm, idx_hbm, out_hbm, idx_v, *rest):
        bufs = rest[:_NBUF]
        gsems = rest[_NBUF:2 * _NBUF]
        ssems = rest[2 * _NBUF:]
        wid = lax.axis_index("s") * NC + lax.axis_index("c")
        src0 = jnp.minimum(wid * span, last_span)
        pltpu.sync_copy(idx_hbm.at[pl.ds(src0, span)], idx_v)

        def chunk_dst(j):
            return jnp.minimum(wid * span + j * _CHUNK, last_base)

        gathers = [None] * _NBUF
        scatters = [None] * _NBUF

        def start_gather(j):
            b = j % _NBUF
            if scatters[b] is not None:
                scatters[b].wait()
                scatters[b] = None
            idx_chunk = idx_v.at[pl.ds(chunk_dst(j) - src0, _CHUNK)]
            gathers[b] = pltpu.async_copy(table_hbm.at[idx_chunk], bufs[b],
                                          gsems[b])

        for j in range(min(_NBUF - 1, per_w)):
            start_gather(j)
        for j in range(per_w):
            b = j % _NBUF
            gathers[b].wait()
            scatters[b] = pltpu.async_copy(
                bufs[b], out_hbm.at[pl.ds(chunk_dst(j), _CHUNK)], ssems[b])
            nxt = j + _NBUF - 1
            if nxt < per_w:
                start_gather(nxt)
        for s in scatters:
            if s is not None:
                s.wait()

    return gather_kernel


def kernel(x, cluster, depth):
    B0 = cluster.shape[0]
    V, D = x.shape
    NC, NS = _sc_geometry()
    NW = NC * NS
    n_real = -(-B0 // _CHUNK)
    n_chunks = -(-n_real // NW) * NW
    idx = cluster.astype(jnp.int32)
    fn = _build_gather(V, D, B0, n_chunks, NC, NS)
    out = lax.cond(
        depth != 0,
        lambda: fn(x, idx),
        lambda: jnp.zeros((B0, D), jnp.float32),
    )
    return out


# speedup vs baseline: 4.2113x; 1.0418x over previous
import functools

import jax
import jax.numpy as jnp
from jax import lax
from jax.experimental import pallas as pl
from jax.experimental.pallas import tpu as pltpu
from jax.experimental.pallas import tpu_sc as plsc

_CHUNK = 128
_NBUF = 6


def _sc_geometry():
    try:
        info = plsc.get_sparse_core_info()
        return info.num_cores, info.num_subcores
    except Exception:
        return 2, 16


@functools.lru_cache(maxsize=None)
def _build_gather(V, D, B0, n_chunks, NC, NS):
    NW = NC * NS
    per_w = n_chunks // NW
    span = per_w * _CHUNK
    last_base = B0 - _CHUNK
    last_span = B0 - span
    mesh = plsc.VectorSubcoreMesh(core_axis_name="c", subcore_axis_name="s")

    @functools.partial(
        pl.kernel,
        mesh=mesh,
        out_type=jax.ShapeDtypeStruct((B0, D), jnp.float32),
        scratch_types=(
            [pltpu.VMEM((span,), jnp.int32)]
            + [pltpu.VMEM((_CHUNK, D), jnp.float32) for _ in range(_NBUF)]
            + [pltpu.SemaphoreType.DMA for _ in range(2 * _NBUF)]
        ),
    )
    def gather_kernel(table_hbm, idx_hbm, out_hbm, idx_v, *rest):
        bufs = rest[:_NBUF]
        gsems = rest[_NBUF:2 * _NBUF]
        ssems = rest[2 * _NBUF:]
        wid = lax.axis_index("s") * NC + lax.axis_index("c")
        src0 = jnp.minimum(wid * span, last_span)
        pltpu.sync_copy(idx_hbm.at[pl.ds(src0, span)], idx_v)

        def chunk_dst(j):
            return jnp.minimum(wid * span + j * _CHUNK, last_base)

        gathers = [None] * _NBUF
        scatters = [None] * _NBUF

        def start_gather(j):
            b = j % _NBUF
            if scatters[b] is not None:
                scatters[b].wait()
                scatters[b] = None
            idx_chunk = idx_v.at[pl.ds(chunk_dst(j) - src0, _CHUNK)]
            gathers[b] = pltpu.async_copy(table_hbm.at[idx_chunk], bufs[b],
                                          gsems[b])

        for j in range(min(_NBUF - 1, per_w)):
            start_gather(j)
        for j in range(per_w):
            b = j % _NBUF
            gathers[b].wait()
            scatters[b] = pltpu.async_copy(
                bufs[b], out_hbm.at[pl.ds(chunk_dst(j), _CHUNK)], ssems[b])
            nxt = j + _NBUF - 1
            if nxt < per_w:
                start_gather(nxt)
        for s in scatters:
            if s is not None:
                s.wait()

    return gather_kernel


def kernel(x, cluster, depth):
    B0 = cluster.shape[0]
    V, D = x.shape
    NC, NS = _sc_geometry()
    NW = NC * NS
    n_real = -(-B0 // _CHUNK)
    n_chunks = -(-n_real // NW) * NW
    idx = cluster.astype(jnp.int32)
    fn = _build_gather(V, D, B0, n_chunks, NC, NS)
    out = lax.cond(
        depth != 0,
        lambda: fn(x, idx),
        lambda: jnp.zeros((B0, D), jnp.float32),
    )
    return out
